# baseline (speedup 1.0000x reference)
import functools
import math

import numpy as np

import jax
import jax.numpy as jnp
from jax.experimental import pallas as pl
from jax.experimental.pallas import tpu as pltpu


def _interp_arrays(out_size, in_size):
    if out_size == 1:
        src = np.zeros((1,), np.float32)
    else:
        src = np.arange(out_size, dtype=np.float32) * np.float32(
            (in_size - 1) / (out_size - 1)
        )
    i0 = np.clip(np.floor(src).astype(np.int32), 0, in_size - 1)
    i1 = np.minimum(i0 + 1, in_size - 1)
    w1 = src - i0.astype(np.float32)
    w0 = np.float32(1.0) - w1
    return i0, i1, w0, w1


def _interp_matrix(out_size, in_size):
    i0, i1, w0, w1 = _interp_arrays(out_size, in_size)
    m = np.zeros((out_size, in_size), np.float32)
    m[np.arange(out_size), i0] += w0
    m[np.arange(out_size), i1] += w1
    return m


def _interleaved_weights(out_size, in_size, scale):
    i0, i1, w0, w1 = _interp_arrays(out_size, in_size)
    j = np.arange(out_size)
    assert np.all((i0 == 2 * j) | (i0 == 2 * j + 1))
    assert np.all((i1 == 2 * j) | (i1 == 2 * j + 1))
    ch = np.zeros((in_size,), np.float32)
    np.add.at(ch, i0, w0)
    np.add.at(ch, i1, w1)
    return np.float32(scale) * ch


def _resize_kernel(x_hbm, wwt_ref, ch_ref, o_hbm,
                   x_buf, z_buf, o_buf, in_sem, out_sem, *, chunk):
    core = pl.program_id(0)
    nsl = x_hbm.shape[0] // pl.num_programs(0)
    half = nsl // 2
    base = core * nsl
    h, w = x_hbm.shape[1], x_hbm.shape[2]
    ho = h // 2
    nch = half // chunk

    wwt = wwt_ref[...].astype(jnp.bfloat16)
    ch = ch_ref[...]
    wo = wwt.shape[1]

    def dma_in(part):
        pltpu.make_async_copy(x_hbm.at[pl.ds(base + part * half, half)],
                              x_buf.at[part], in_sem.at[part]).start()

    def wait_in(part):
        pltpu.make_async_copy(x_hbm.at[pl.ds(base, half)],
                              x_buf.at[part], in_sem.at[part]).wait()

    dma_in(0)
    dma_in(1)
    for part in range(2):
        wait_in(part)
        for k in range(nch):
            x = x_buf[part, pl.ds(k * chunk, chunk)].astype(jnp.bfloat16)
            v = jnp.dot(x.reshape(chunk * h, w), wwt,
                        preferred_element_type=jnp.float32,
                        ).reshape(chunk, h, wo)
            y = ch * v
            z_buf[pl.ds(0, chunk)] = y + pltpu.roll(y, h - 1, 1)
            g = part * half + k * chunk
            o_buf[pl.ds(g, chunk)] = z_buf[pl.ds(0, chunk),
                                           pl.Slice(0, ho, 2), :]
            pltpu.make_async_copy(
                o_buf.at[pl.ds(g, chunk)],
                o_hbm.at[pl.ds(base + g, chunk)],
                out_sem.at[part * nch + k]).start()
    for k in range(2 * nch):
        pltpu.make_async_copy(
            o_buf.at[pl.ds(k * chunk, chunk)],
            o_hbm.at[pl.ds(base + k * chunk, chunk)],
            out_sem.at[k]).wait()


def kernel(x):
    vel_resize = 2.0
    factor = 1.0 / vel_resize
    N, C, H, W = x.shape
    H_out = int(math.floor(H * factor))
    W_out = int(math.floor(W * factor))
    assert H == 2 * H_out and W == 2 * W_out
    B = N * C
    assert B % 4 == 0

    wwt = jnp.asarray(np.ascontiguousarray(_interp_matrix(W_out, W).T))
    ch = jnp.asarray(_interleaved_weights(H_out, H, factor)).reshape(1, H, 1)

    nsl = B // 2
    half = nsl // 2
    chunk = 2
    while chunk > 1 and half % chunk:
        chunk //= 2
    nch = half // chunk

    body = functools.partial(_resize_kernel, chunk=chunk)
    out3 = pl.pallas_call(
        body,
        out_shape=jax.ShapeDtypeStruct((B, H_out, W_out), x.dtype),
        grid=(2,),
        in_specs=[
            pl.BlockSpec(memory_space=pl.ANY),
            pl.BlockSpec((W, W_out), lambda c: (0, 0)),
            pl.BlockSpec((1, H, 1), lambda c: (0, 0, 0)),
        ],
        out_specs=pl.BlockSpec(memory_space=pl.ANY),
        scratch_shapes=[
            pltpu.VMEM((2, half, H, W), jnp.float32),
            pltpu.VMEM((chunk, H, W_out), jnp.float32),
            pltpu.VMEM((nsl, H_out, W_out), jnp.float32),
            pltpu.SemaphoreType.DMA((2,)),
            pltpu.SemaphoreType.DMA((2 * nch,)),
        ],
        compiler_params=pltpu.CompilerParams(
            dimension_semantics=("parallel",),
            vmem_limit_bytes=int(64 * 1024 * 1024 * 0.85),
        ),
    )(x.reshape(B, H, W), wwt, ch)
    return out3.reshape(N, C, H_out, W_out)


# speedup vs baseline: 1.1536x; 1.1536x over previous
import functools
import math

import numpy as np

import jax
import jax.numpy as jnp
from jax.experimental import pallas as pl
from jax.experimental.pallas import tpu as pltpu


def _interp_arrays(out_size, in_size):
    if out_size == 1:
        src = np.zeros((1,), np.float32)
    else:
        src = np.arange(out_size, dtype=np.float32) * np.float32(
            (in_size - 1) / (out_size - 1)
        )
    i0 = np.clip(np.floor(src).astype(np.int32), 0, in_size - 1)
    i1 = np.minimum(i0 + 1, in_size - 1)
    w1 = src - i0.astype(np.float32)
    w0 = np.float32(1.0) - w1
    return i0, i1, w0, w1


def _interp_matrix(out_size, in_size):
    i0, i1, w0, w1 = _interp_arrays(out_size, in_size)
    m = np.zeros((out_size, in_size), np.float32)
    m[np.arange(out_size), i0] += w0
    m[np.arange(out_size), i1] += w1
    return m


def _interleaved_weights(out_size, in_size, scale):
    i0, i1, w0, w1 = _interp_arrays(out_size, in_size)
    j = np.arange(out_size)
    assert np.all((i0 == 2 * j) | (i0 == 2 * j + 1))
    assert np.all((i1 == 2 * j) | (i1 == 2 * j + 1))
    ch = np.zeros((in_size,), np.float32)
    np.add.at(ch, i0, w0)
    np.add.at(ch, i1, w1)
    return np.float32(scale) * ch


def _resize_kernel(x_ref, wwt_ref, ch_ref, o_hbm, z_buf, o_buf, out_sem, *,
                   chunk):
    core = pl.program_id(0)
    nsl, h, w = x_ref.shape
    base = core * nsl
    nch = nsl // chunk
    ho = h // 2

    wwt = wwt_ref[...].astype(jnp.bfloat16)
    ch = ch_ref[...]
    wo = wwt.shape[1]

    for k in range(nch):
        x = x_ref[pl.ds(k * chunk, chunk)].astype(jnp.bfloat16)
        v = jnp.dot(x.reshape(chunk * h, w), wwt,
                    preferred_element_type=jnp.float32).reshape(chunk, h, wo)
        y = ch * v
        z_buf[pl.ds(0, chunk)] = y + pltpu.roll(y, h - 1, 1)
        o_buf[pl.ds(k * chunk, chunk)] = z_buf[pl.ds(0, chunk),
                                               pl.Slice(0, ho, 2), :]
        pltpu.make_async_copy(
            o_buf.at[pl.ds(k * chunk, chunk)],
            o_hbm.at[pl.ds(base + k * chunk, chunk)],
            out_sem.at[k]).start()
    for k in range(nch):
        pltpu.make_async_copy(
            o_buf.at[pl.ds(k * chunk, chunk)],
            o_hbm.at[pl.ds(base + k * chunk, chunk)],
            out_sem.at[k]).wait()


def kernel(x):
    vel_resize = 2.0
    factor = 1.0 / vel_resize
    N, C, H, W = x.shape
    H_out = int(math.floor(H * factor))
    W_out = int(math.floor(W * factor))
    assert H == 2 * H_out and W == 2 * W_out
    B = N * C
    assert B % 2 == 0

    wwt = jnp.asarray(np.ascontiguousarray(_interp_matrix(W_out, W).T))
    ch = jnp.asarray(_interleaved_weights(H_out, H, factor)).reshape(1, H, 1)

    nsl = B // 2
    chunk = 1
    nch = nsl // chunk

    body = functools.partial(_resize_kernel, chunk=chunk)
    out3 = pl.pallas_call(
        body,
        out_shape=jax.ShapeDtypeStruct((B, H_out, W_out), x.dtype),
        grid=(2,),
        in_specs=[
            pl.BlockSpec((nsl, H, W), lambda c: (c, 0, 0)),
            pl.BlockSpec((W, W_out), lambda c: (0, 0)),
            pl.BlockSpec((1, H, 1), lambda c: (0, 0, 0)),
        ],
        out_specs=pl.BlockSpec(memory_space=pl.ANY),
        scratch_shapes=[
            pltpu.VMEM((chunk, H, W_out), jnp.float32),
            pltpu.VMEM((nsl, H_out, W_out), jnp.float32),
            pltpu.SemaphoreType.DMA((nch,)),
        ],
        compiler_params=pltpu.CompilerParams(
            dimension_semantics=("parallel",),
            vmem_limit_bytes=int(64 * 1024 * 1024 * 0.85),
        ),
    )(x.reshape(B, H, W), wwt, ch)
    return out3.reshape(N, C, H_out, W_out)
